# baseline (speedup 1.0000x reference)
import jax
import jax.numpy as jnp
from jax import lax
from jax.experimental import pallas as pl
from jax.experimental.pallas import tpu as pltpu
from jax.experimental.pallas import tpu_sc as plsc

B = 1024
L = 200
A = 8
DW = 300
DP = 50

NW = 32
RPW = (B * L) // NW
CHUNK = 128
NCH = RPW // CHUNK
SCHUNK = 320
NCHS = RPW // SCHUNK
ABATCH = B // NW
WBLK = 3

NPOS = 50
NPOLAR = 4
NPOSITION = 201
POS_REP = 128
POLAR_REP = 1024
POSITION_REP = 64


def _sc_body(ctx_i, pos_i, polar_i, position_i, asp_i, bnd_i,
             wt_phys, pt_phys, plt_phys, pst_phys,
             o_word, o_pos, o_polar, o_position, o_aspect,
             idx_v, idx3A, idx3B, bufA, bufB, bnd_v, len_v, apool_v,
             gsemA, gsemB, wsemA, wsemB):
    wid = lax.axis_index("s") * 2 + lax.axis_index("c")
    base = wid * RPW
    lanes = lax.iota(jnp.int32, 16)

    def build(idx3, c):
        off = c * CHUNK
        for t in range((CHUNK * WBLK) // 16):
            e = t * 16 + lanes
            k = e // WBLK
            cc = e - k * WBLK
            src = plsc.load_gather(idx_v, [off + k])
            idx3[pl.ds(t * 16, 16)] = src * WBLK + cc

    def wg(idx3, buf, sem):
        pltpu.async_copy(wt_phys.at[idx3], buf, sem)

    def wg_wait(idx3, buf, sem):
        pltpu.make_async_copy(wt_phys.at[idx3], buf, sem).wait()

    def ww(buf, c, sem):
        pltpu.async_copy(
            buf, o_word.at[pl.ds(WBLK * (base + c * CHUNK), WBLK * CHUNK)],
            sem)

    def ww_wait(buf, c, sem):
        pltpu.make_async_copy(
            buf, o_word.at[pl.ds(WBLK * (base + c * CHUNK), WBLK * CHUNK)],
            sem).wait()

    pltpu.sync_copy(ctx_i.at[pl.ds(base, RPW)], idx_v)
    build(idx3A, 0)
    wg(idx3A, bufA, gsemA)
    build(idx3B, 1)
    wg(idx3B, bufB, gsemB)
    wg_wait(idx3A, bufA, gsemA)
    ww(bufA, 0, wsemA)

    def wbody(i, carry):
        c = 2 * i
        build(idx3A, c)
        ww_wait(bufA, c - 2, wsemA)
        wg(idx3A, bufA, gsemA)
        wg_wait(idx3B, bufB, gsemB)
        ww(bufB, c - 1, wsemB)
        build(idx3B, c + 1)
        ww_wait(bufB, c - 1, wsemB)
        wg(idx3B, bufB, gsemB)
        wg_wait(idx3A, bufA, gsemA)
        ww(bufA, c, wsemA)
        return carry

    lax.fori_loop(1, NCH // 2, wbody, 0)
    wg_wait(idx3B, bufB, gsemB)
    ww(bufB, NCH - 1, wsemB)
    ww_wait(bufA, NCH - 2, wsemA)
    ww_wait(bufB, NCH - 1, wsemB)

    def small_phase(idx_hbm, table, out, n_rows, rep_mask, sidxA, sidxB):
        pltpu.sync_copy(idx_hbm.at[pl.ds(base, RPW)], idx_v)
        sbufA = bufA.at[pl.ds(0, SCHUNK)]
        sbufB = bufB.at[pl.ds(0, SCHUNK)]

        def sbuild(sidx, c):
            for t in range(SCHUNK // 16):
                iv = idx_v[pl.ds(c * SCHUNK + t * 16, 16)]
                spread = (t * 16 + lanes) & rep_mask
                sidx[pl.ds(t * 16, 16)] = iv + n_rows * spread

        def sg(sidx, slot, sem):
            pltpu.async_copy(table.at[sidx.at[pl.ds(0, SCHUNK)]], slot, sem)

        def sg_wait(sidx, slot, sem):
            pltpu.make_async_copy(
                table.at[sidx.at[pl.ds(0, SCHUNK)]], slot, sem).wait()

        def sw(slot, c, sem):
            pltpu.async_copy(slot, out.at[pl.ds(base + c * SCHUNK, SCHUNK)],
                             sem)

        def sw_wait(slot, c, sem):
            pltpu.make_async_copy(
                slot, out.at[pl.ds(base + c * SCHUNK, SCHUNK)], sem).wait()

        sbuild(sidxA, 0)
        sg(sidxA, sbufA, gsemA)
        sbuild(sidxB, 1)
        sg(sidxB, sbufB, gsemB)
        sg_wait(sidxA, sbufA, gsemA)
        sw(sbufA, 0, wsemA)

        def body(i, carry):
            c = 2 * i
            sbuild(sidxA, c)
            sw_wait(sbufA, c - 2, wsemA)
            sg(sidxA, sbufA, gsemA)
            sg_wait(sidxB, sbufB, gsemB)
            sw(sbufB, c - 1, wsemB)
            sbuild(sidxB, c + 1)
            sw_wait(sbufB, c - 1, wsemB)
            sg(sidxB, sbufB, gsemB)
            sg_wait(sidxA, sbufA, gsemA)
            sw(sbufA, c, wsemA)
            return carry

        lax.fori_loop(1, NCHS // 2, body, 0)
        sg_wait(sidxB, sbufB, gsemB)
        sw(sbufB, NCHS - 1, wsemB)
        sw_wait(sbufA, NCHS - 2, wsemA)
        sw_wait(sbufB, NCHS - 1, wsemB)

    small_phase(pos_i, pt_phys, o_pos, NPOS, POS_REP - 1, idx3A, idx3B)
    small_phase(polar_i, plt_phys, o_polar, NPOLAR, POLAR_REP - 1,
                idx3A, idx3B)
    small_phase(position_i, pst_phys, o_position, NPOSITION,
                POSITION_REP - 1, idx3A, idx3B)

    abase = wid * ABATCH
    pltpu.sync_copy(bnd_i.at[pl.ds(abase * 2, 2 * ABATCH)], bnd_v)
    for sub in range(2):
        bsub = abase + sub * 16
        pltpu.sync_copy(asp_i.at[pl.ds(bsub * A, 16 * A)],
                        idx_v.at[pl.ds(0, 16 * A)])
        build(idx3A, 0)
        wg(idx3A, bufA, gsemA)
        wg_wait(idx3A, bufA, gsemA)
        bidx = (sub * 16 + lanes) * 2
        b0 = plsc.load_gather(bnd_v, [bidx])
        b1 = plsc.load_gather(bnd_v, [bidx + 1])
        len_v[...] = (b1 - b0 + 1).astype(jnp.float32)

        def achunk(k, carry):
            j = k * 16 + lanes
            bl = j // DW
            dd = j - bl * DW
            blk = dd // 128
            col = dd - blk * 128
            acc = plsc.load_gather(bufA, [(bl * A) * WBLK + blk, col])
            for a in range(1, A):
                acc = acc + plsc.load_gather(
                    bufA, [(bl * A + a) * WBLK + blk, col])
            lenv = plsc.load_gather(len_v, [bl])
            plsc.store_scatter(apool_v, [bl * WBLK + blk, col], acc / lenv)
            return carry

        lax.fori_loop(0, (16 * DW) // 16, achunk, 0)
        pltpu.sync_copy(apool_v, o_aspect.at[pl.ds(WBLK * bsub, WBLK * 16)])


def kernel(word_table, pos_table, polar_table, position_table,
           context_indices, pos_indices, polar_indices, text_indices,
           position_indices, aspect_indices, aspect_boundary, target, len_s):
    one = lax.optimization_barrier(jnp.ones((), jnp.float32))
    wt_phys = jnp.pad(word_table * one, ((0, 0), (0, WBLK * 128 - DW))) \
                 .reshape(word_table.shape[0] * WBLK, 128)
    pt_phys = jnp.tile(jnp.pad(pos_table * one, ((0, 0), (0, 128 - DP))),
                       (POS_REP, 1))
    plt_phys = jnp.tile(jnp.pad(polar_table * one, ((0, 0), (0, 128 - DP))),
                        (POLAR_REP, 1))
    pst_phys = jnp.tile(jnp.pad(position_table * one, ((0, 0), (0, 128 - DP))),
                        (POSITION_REP, 1))

    ctx = context_indices.reshape(-1)
    posi = pos_indices.reshape(-1)
    poli = polar_indices.reshape(-1)
    psni = position_indices.reshape(-1)
    aspi = aspect_indices.reshape(-1)
    bnd = aspect_boundary.reshape(-1)

    k = pl.kernel(
        _sc_body,
        out_type=(
            jax.ShapeDtypeStruct((B * L * WBLK, 128), jnp.float32),
            jax.ShapeDtypeStruct((B * L, 128), jnp.float32),
            jax.ShapeDtypeStruct((B * L, 128), jnp.float32),
            jax.ShapeDtypeStruct((B * L, 128), jnp.float32),
            jax.ShapeDtypeStruct((B * WBLK, 128), jnp.float32),
        ),
        mesh=plsc.VectorSubcoreMesh(core_axis_name="c", subcore_axis_name="s"),
        scratch_types=[
            pltpu.VMEM((RPW,), jnp.int32),
            pltpu.VMEM((CHUNK * WBLK,), jnp.int32),
            pltpu.VMEM((CHUNK * WBLK,), jnp.int32),
            pltpu.VMEM((CHUNK * WBLK, 128), jnp.float32),
            pltpu.VMEM((CHUNK * WBLK, 128), jnp.float32),
            pltpu.VMEM((2 * ABATCH,), jnp.int32),
            pltpu.VMEM((16,), jnp.float32),
            pltpu.VMEM((WBLK * 16, 128), jnp.float32),
            pltpu.SemaphoreType.DMA,
            pltpu.SemaphoreType.DMA,
            pltpu.SemaphoreType.DMA,
            pltpu.SemaphoreType.DMA,
        ],
        compiler_params=pltpu.CompilerParams(
            needs_layout_passes=False, use_tc_tiling_on_sc=False
        ),
    )
    o_word, o_pos, o_polar, o_position, o_aspect = k(
        ctx, posi, poli, psni, aspi, bnd,
        wt_phys, pt_phys, plt_phys, pst_phys,
    )
    word = (o_word.reshape(B * L, WBLK * 128)[:, :DW] * one) \
        .reshape(B, L, DW)
    pos = (o_pos[:, :DP] * one).reshape(B, L, DP)
    polar = (o_polar[:, :DP] * one).reshape(B, L, DP)
    position = (o_position[:, :DP] * one).reshape(B, L, DP)
    aspect_pool = (o_aspect.reshape(B, WBLK * 128)[:, :DW] * one) \
        .reshape(B, 1, DW)
    return (aspect_pool, word, pos, polar, position)


# speedup vs baseline: 1.5580x; 1.5580x over previous
import jax
import jax.numpy as jnp
from jax import lax
from jax.experimental import pallas as pl
from jax.experimental.pallas import tpu as pltpu
from jax.experimental.pallas import tpu_sc as plsc

B = 1024
L = 200
A = 8
DW = 300
DP = 50

NW = 32
RPW = (B * L) // NW
CHUNK = 128
NCH = RPW // CHUNK
SCHUNK = 320
NCHS = RPW // SCHUNK
ABATCH = B // NW
WBLK = 3

NPOS = 50
NPOLAR = 4
NPOSITION = 201
POS_REP = 128
POLAR_REP = 1024
POSITION_REP = 64


def _sc_body(idx_all, wt_phys, small_t,
             o_word, o_small, o_aspect,
             idx_v, idx3A, idx3B, bufA, bufB, bnd_v, len_v, apool_v,
             gsemA, gsemB, wsemA, wsemB):
    wid = lax.axis_index("s") * 2 + lax.axis_index("c")
    base = wid * RPW
    lanes = lax.iota(jnp.int32, 16)

    def build(idx3, c):
        off = c * CHUNK
        for t in range((CHUNK * WBLK) // 16):
            e = t * 16 + lanes
            k = e // WBLK
            cc = e - k * WBLK
            src = plsc.load_gather(idx_v, [off + k])
            idx3[pl.ds(t * 16, 16)] = src * WBLK + cc

    def wg(idx3, buf, sem):
        pltpu.async_copy(wt_phys.at[idx3], buf, sem)

    def wg_wait(idx3, buf, sem):
        pltpu.make_async_copy(wt_phys.at[idx3], buf, sem).wait()

    def ww(buf, c, sem):
        pltpu.async_copy(
            buf, o_word.at[pl.ds(WBLK * (base + c * CHUNK), WBLK * CHUNK)],
            sem)

    def ww_wait(buf, c, sem):
        pltpu.make_async_copy(
            buf, o_word.at[pl.ds(WBLK * (base + c * CHUNK), WBLK * CHUNK)],
            sem).wait()

    pltpu.sync_copy(idx_all.at[pl.ds(base, RPW)], idx_v)
    build(idx3A, 0)
    wg(idx3A, bufA, gsemA)
    build(idx3B, 1)
    wg(idx3B, bufB, gsemB)
    wg_wait(idx3A, bufA, gsemA)
    ww(bufA, 0, wsemA)

    def wbody(i, carry):
        c = 2 * i
        build(idx3A, c)
        ww_wait(bufA, c - 2, wsemA)
        wg(idx3A, bufA, gsemA)
        wg_wait(idx3B, bufB, gsemB)
        ww(bufB, c - 1, wsemB)
        build(idx3B, c + 1)
        ww_wait(bufB, c - 1, wsemB)
        wg(idx3B, bufB, gsemB)
        wg_wait(idx3A, bufA, gsemA)
        ww(bufA, c, wsemA)
        return carry

    lax.fori_loop(1, NCH // 2, wbody, 0)
    wg_wait(idx3B, bufB, gsemB)
    ww(bufB, NCH - 1, wsemB)
    ww_wait(bufA, NCH - 2, wsemA)
    ww_wait(bufB, NCH - 1, wsemB)

    def small_phase(idx_off, tbl_base, out_base, n_rows, rep_mask,
                    sidxA, sidxB):
        pltpu.sync_copy(idx_all.at[pl.ds(idx_off + base, RPW)], idx_v)
        sbufA = bufA.at[pl.ds(0, SCHUNK)]
        sbufB = bufB.at[pl.ds(0, SCHUNK)]

        def sbuild(sidx, c):
            for t in range(SCHUNK // 16):
                iv = idx_v[pl.ds(c * SCHUNK + t * 16, 16)]
                spread = (t * 16 + lanes) & rep_mask
                sidx[pl.ds(t * 16, 16)] = tbl_base + iv + n_rows * spread

        def sg(sidx, slot, sem):
            pltpu.async_copy(small_t.at[sidx.at[pl.ds(0, SCHUNK)]], slot, sem)

        def sg_wait(sidx, slot, sem):
            pltpu.make_async_copy(
                small_t.at[sidx.at[pl.ds(0, SCHUNK)]], slot, sem).wait()

        def sw(slot, c, sem):
            pltpu.async_copy(
                slot, o_small.at[pl.ds(out_base + base + c * SCHUNK, SCHUNK)],
                sem)

        def sw_wait(slot, c, sem):
            pltpu.make_async_copy(
                slot, o_small.at[pl.ds(out_base + base + c * SCHUNK, SCHUNK)],
                sem).wait()

        sbuild(sidxA, 0)
        sg(sidxA, sbufA, gsemA)
        sbuild(sidxB, 1)
        sg(sidxB, sbufB, gsemB)
        sg_wait(sidxA, sbufA, gsemA)
        sw(sbufA, 0, wsemA)

        def body(i, carry):
            c = 2 * i
            sbuild(sidxA, c)
            sw_wait(sbufA, c - 2, wsemA)
            sg(sidxA, sbufA, gsemA)
            sg_wait(sidxB, sbufB, gsemB)
            sw(sbufB, c - 1, wsemB)
            sbuild(sidxB, c + 1)
            sw_wait(sbufB, c - 1, wsemB)
            sg(sidxB, sbufB, gsemB)
            sg_wait(sidxA, sbufA, gsemA)
            sw(sbufA, c, wsemA)
            return carry

        lax.fori_loop(1, NCHS // 2, body, 0)
        sg_wait(sidxB, sbufB, gsemB)
        sw(sbufB, NCHS - 1, wsemB)
        sw_wait(sbufA, NCHS - 2, wsemA)
        sw_wait(sbufB, NCHS - 1, wsemB)

    small_phase(1 * B * L, 0, 0, NPOS, POS_REP - 1, idx3A, idx3B)
    small_phase(2 * B * L, NPOS * POS_REP, B * L, NPOLAR, POLAR_REP - 1,
                idx3A, idx3B)
    small_phase(3 * B * L, NPOS * POS_REP + NPOLAR * POLAR_REP, 2 * B * L,
                NPOSITION, POSITION_REP - 1, idx3A, idx3B)

    abase = wid * ABATCH
    pltpu.sync_copy(
        idx_all.at[pl.ds(4 * B * L + B * A + abase * 2, 2 * ABATCH)], bnd_v)
    for sub in range(2):
        bsub = abase + sub * 16
        pltpu.sync_copy(idx_all.at[pl.ds(4 * B * L + bsub * A, 16 * A)],
                        idx_v.at[pl.ds(0, 16 * A)])
        build(idx3A, 0)
        wg(idx3A, bufA, gsemA)
        wg_wait(idx3A, bufA, gsemA)
        bidx = (sub * 16 + lanes) * 2
        b0 = plsc.load_gather(bnd_v, [bidx])
        b1 = plsc.load_gather(bnd_v, [bidx + 1])
        len_v[...] = (b1 - b0 + 1).astype(jnp.float32)

        def achunk(k, carry):
            j = k * 16 + lanes
            bl = j // DW
            dd = j - bl * DW
            blk = dd // 128
            col = dd - blk * 128
            acc = plsc.load_gather(bufA, [(bl * A) * WBLK + blk, col])
            for a in range(1, A):
                acc = acc + plsc.load_gather(
                    bufA, [(bl * A + a) * WBLK + blk, col])
            lenv = plsc.load_gather(len_v, [bl])
            plsc.store_scatter(apool_v, [bl * WBLK + blk, col], acc / lenv)
            return carry

        lax.fori_loop(0, (16 * DW) // 16, achunk, 0)
        pltpu.sync_copy(apool_v, o_aspect.at[pl.ds(WBLK * bsub, WBLK * 16)])


def kernel(word_table, pos_table, polar_table, position_table,
           context_indices, pos_indices, polar_indices, text_indices,
           position_indices, aspect_indices, aspect_boundary, target, len_s):
    wt_phys = jnp.pad(word_table, ((0, 0), (0, WBLK * 128 - DW))) \
                 .reshape(word_table.shape[0] * WBLK, 128)
    small_t = jnp.concatenate([
        jnp.tile(jnp.pad(pos_table, ((0, 0), (0, 128 - DP))), (POS_REP, 1)),
        jnp.tile(jnp.pad(polar_table, ((0, 0), (0, 128 - DP))),
                 (POLAR_REP, 1)),
        jnp.tile(jnp.pad(position_table, ((0, 0), (0, 128 - DP))),
                 (POSITION_REP, 1)),
    ])
    idx_all = jnp.concatenate([
        context_indices.reshape(-1), pos_indices.reshape(-1),
        polar_indices.reshape(-1), position_indices.reshape(-1),
        aspect_indices.reshape(-1), aspect_boundary.reshape(-1),
    ])

    k = pl.kernel(
        _sc_body,
        out_type=(
            jax.ShapeDtypeStruct((B * L * WBLK, 128), jnp.float32),
            jax.ShapeDtypeStruct((3 * B * L, 128), jnp.float32),
            jax.ShapeDtypeStruct((B * WBLK, 128), jnp.float32),
        ),
        mesh=plsc.VectorSubcoreMesh(core_axis_name="c", subcore_axis_name="s"),
        scratch_types=[
            pltpu.VMEM((RPW,), jnp.int32),
            pltpu.VMEM((CHUNK * WBLK,), jnp.int32),
            pltpu.VMEM((CHUNK * WBLK,), jnp.int32),
            pltpu.VMEM((CHUNK * WBLK, 128), jnp.float32),
            pltpu.VMEM((CHUNK * WBLK, 128), jnp.float32),
            pltpu.VMEM((2 * ABATCH,), jnp.int32),
            pltpu.VMEM((16,), jnp.float32),
            pltpu.VMEM((WBLK * 16, 128), jnp.float32),
            pltpu.SemaphoreType.DMA,
            pltpu.SemaphoreType.DMA,
            pltpu.SemaphoreType.DMA,
            pltpu.SemaphoreType.DMA,
        ],
        compiler_params=pltpu.CompilerParams(
            needs_layout_passes=False, use_tc_tiling_on_sc=False
        ),
    )
    o_word, o_small, o_aspect = k(idx_all, wt_phys, small_t)
    word = o_word.reshape(B * L, WBLK * 128)[:, :DW].reshape(B, L, DW)
    pos = o_small[0 * B * L:1 * B * L, :DP].reshape(B, L, DP)
    polar = o_small[1 * B * L:2 * B * L, :DP].reshape(B, L, DP)
    position = o_small[2 * B * L:3 * B * L, :DP].reshape(B, L, DP)
    aspect_pool = o_aspect.reshape(B, WBLK * 128)[:, :DW].reshape(B, 1, DW)
    return (aspect_pool, word, pos, polar, position)


# speedup vs baseline: 1.7270x; 1.1085x over previous
import jax
import jax.numpy as jnp
from jax import lax
from jax.experimental import pallas as pl
from jax.experimental.pallas import tpu as pltpu
from jax.experimental.pallas import tpu_sc as plsc

B = 1024
L = 200
A = 8
DW = 300
DP = 50

NW = 32
RPW = (B * L) // NW
CHUNK = 128
NCH = RPW // CHUNK
SCHUNK = 320
NCHS = RPW // SCHUNK
ABATCH = B // NW
WBLK = 3

NPOS = 50
NPOLAR = 4
NPOSITION = 201
POS_REP = 256
POLAR_REP = 2048
POSITION_REP = 128


def _sc_body(ctx_i, pos_i, polar_i, position_i, asp_i, bnd_i,
             wt_phys, pt_phys, plt_phys, pst_phys,
             o_word, o_pos, o_polar, o_position, o_aspect,
             idx_v, idx3A, idx3B, bufA, bufB, bnd_v, len_v, apool_v,
             gsemA, gsemB, wsemA, wsemB):
    wid = lax.axis_index("s") * 2 + lax.axis_index("c")
    base = wid * RPW
    lanes = lax.iota(jnp.int32, 16)

    def build(idx3, c):
        off = c * CHUNK
        for t in range((CHUNK * WBLK) // 16):
            e = t * 16 + lanes
            k = e // WBLK
            cc = e - k * WBLK
            src = plsc.load_gather(idx_v, [off + k])
            idx3[pl.ds(t * 16, 16)] = src * WBLK + cc

    def wg(idx3, buf, sem):
        pltpu.async_copy(wt_phys.at[idx3], buf, sem)

    def wg_wait(idx3, buf, sem):
        pltpu.make_async_copy(wt_phys.at[idx3], buf, sem).wait()

    def ww(buf, c, sem):
        pltpu.async_copy(
            buf, o_word.at[pl.ds(WBLK * (base + c * CHUNK), WBLK * CHUNK)],
            sem)

    def ww_wait(buf, c, sem):
        pltpu.make_async_copy(
            buf, o_word.at[pl.ds(WBLK * (base + c * CHUNK), WBLK * CHUNK)],
            sem).wait()

    pltpu.sync_copy(ctx_i.at[pl.ds(base, RPW)], idx_v)
    build(idx3A, 0)
    wg(idx3A, bufA, gsemA)
    build(idx3B, 1)
    wg(idx3B, bufB, gsemB)
    wg_wait(idx3A, bufA, gsemA)
    ww(bufA, 0, wsemA)

    def wbody(i, carry):
        c = 2 * i
        build(idx3A, c)
        ww_wait(bufA, c - 2, wsemA)
        wg(idx3A, bufA, gsemA)
        wg_wait(idx3B, bufB, gsemB)
        ww(bufB, c - 1, wsemB)
        build(idx3B, c + 1)
        ww_wait(bufB, c - 1, wsemB)
        wg(idx3B, bufB, gsemB)
        wg_wait(idx3A, bufA, gsemA)
        ww(bufA, c, wsemA)
        return carry

    lax.fori_loop(1, NCH // 2, wbody, 0)
    wg_wait(idx3B, bufB, gsemB)
    ww(bufB, NCH - 1, wsemB)
    ww_wait(bufA, NCH - 2, wsemA)
    ww_wait(bufB, NCH - 1, wsemB)

    def small_phase(idx_hbm, table, out, n_rows, rep_mask, sidxA, sidxB):
        pltpu.sync_copy(idx_hbm.at[pl.ds(base, RPW)], idx_v)
        sbufA = bufA.at[pl.ds(0, SCHUNK)]
        sbufB = bufB.at[pl.ds(0, SCHUNK)]

        def sbuild(sidx, c):
            for t in range(SCHUNK // 16):
                iv = idx_v[pl.ds(c * SCHUNK + t * 16, 16)]
                spread = (t * 16 + lanes) & rep_mask
                sidx[pl.ds(t * 16, 16)] = iv + n_rows * spread

        def sg(sidx, slot, sem):
            pltpu.async_copy(table.at[sidx.at[pl.ds(0, SCHUNK)]], slot, sem)

        def sg_wait(sidx, slot, sem):
            pltpu.make_async_copy(
                table.at[sidx.at[pl.ds(0, SCHUNK)]], slot, sem).wait()

        def sw(slot, c, sem):
            pltpu.async_copy(slot, out.at[pl.ds(base + c * SCHUNK, SCHUNK)],
                             sem)

        def sw_wait(slot, c, sem):
            pltpu.make_async_copy(
                slot, out.at[pl.ds(base + c * SCHUNK, SCHUNK)], sem).wait()

        sbuild(sidxA, 0)
        sg(sidxA, sbufA, gsemA)
        sbuild(sidxB, 1)
        sg(sidxB, sbufB, gsemB)
        sg_wait(sidxA, sbufA, gsemA)
        sw(sbufA, 0, wsemA)

        def body(i, carry):
            c = 2 * i
            sbuild(sidxA, c)
            sw_wait(sbufA, c - 2, wsemA)
            sg(sidxA, sbufA, gsemA)
            sg_wait(sidxB, sbufB, gsemB)
            sw(sbufB, c - 1, wsemB)
            sbuild(sidxB, c + 1)
            sw_wait(sbufB, c - 1, wsemB)
            sg(sidxB, sbufB, gsemB)
            sg_wait(sidxA, sbufA, gsemA)
            sw(sbufA, c, wsemA)
            return carry

        lax.fori_loop(1, NCHS // 2, body, 0)
        sg_wait(sidxB, sbufB, gsemB)
        sw(sbufB, NCHS - 1, wsemB)
        sw_wait(sbufA, NCHS - 2, wsemA)
        sw_wait(sbufB, NCHS - 1, wsemB)

    small_phase(pos_i, pt_phys, o_pos, NPOS, POS_REP - 1, idx3A, idx3B)
    small_phase(polar_i, plt_phys, o_polar, NPOLAR, POLAR_REP - 1,
                idx3A, idx3B)
    small_phase(position_i, pst_phys, o_position, NPOSITION,
                POSITION_REP - 1, idx3A, idx3B)

    abase = wid * ABATCH
    pltpu.sync_copy(bnd_i.at[pl.ds(abase * 2, 2 * ABATCH)], bnd_v)
    for sub in range(2):
        bsub = abase + sub * 16
        pltpu.sync_copy(asp_i.at[pl.ds(bsub * A, 16 * A)],
                        idx_v.at[pl.ds(0, 16 * A)])
        build(idx3A, 0)
        wg(idx3A, bufA, gsemA)
        wg_wait(idx3A, bufA, gsemA)
        bidx = (sub * 16 + lanes) * 2
        b0 = plsc.load_gather(bnd_v, [bidx])
        b1 = plsc.load_gather(bnd_v, [bidx + 1])
        len_v[...] = (b1 - b0 + 1).astype(jnp.float32)

        def achunk(k, carry):
            j = k * 16 + lanes
            bl = j // DW
            dd = j - bl * DW
            blk = dd // 128
            col = dd - blk * 128
            acc = plsc.load_gather(bufA, [(bl * A) * WBLK + blk, col])
            for a in range(1, A):
                acc = acc + plsc.load_gather(
                    bufA, [(bl * A + a) * WBLK + blk, col])
            lenv = plsc.load_gather(len_v, [bl])
            plsc.store_scatter(apool_v, [bl * WBLK + blk, col], acc / lenv)
            return carry

        lax.fori_loop(0, (16 * DW) // 16, achunk, 0)
        pltpu.sync_copy(apool_v, o_aspect.at[pl.ds(WBLK * bsub, WBLK * 16)])


def kernel(word_table, pos_table, polar_table, position_table,
           context_indices, pos_indices, polar_indices, text_indices,
           position_indices, aspect_indices, aspect_boundary, target, len_s):
    wt_phys = jnp.pad(word_table, ((0, 0), (0, WBLK * 128 - DW))) \
                 .reshape(word_table.shape[0] * WBLK, 128)
    pt_phys = jnp.tile(jnp.pad(pos_table, ((0, 0), (0, 128 - DP))),
                       (POS_REP, 1))
    plt_phys = jnp.tile(jnp.pad(polar_table, ((0, 0), (0, 128 - DP))),
                        (POLAR_REP, 1))
    pst_phys = jnp.tile(jnp.pad(position_table, ((0, 0), (0, 128 - DP))),
                        (POSITION_REP, 1))

    ctx = context_indices.reshape(-1)
    posi = pos_indices.reshape(-1)
    poli = polar_indices.reshape(-1)
    psni = position_indices.reshape(-1)
    aspi = aspect_indices.reshape(-1)
    bnd = aspect_boundary.reshape(-1)

    k = pl.kernel(
        _sc_body,
        out_type=(
            jax.ShapeDtypeStruct((B * L * WBLK, 128), jnp.float32),
            jax.ShapeDtypeStruct((B * L, 128), jnp.float32),
            jax.ShapeDtypeStruct((B * L, 128), jnp.float32),
            jax.ShapeDtypeStruct((B * L, 128), jnp.float32),
            jax.ShapeDtypeStruct((B * WBLK, 128), jnp.float32),
        ),
        mesh=plsc.VectorSubcoreMesh(core_axis_name="c", subcore_axis_name="s"),
        scratch_types=[
            pltpu.VMEM((RPW,), jnp.int32),
            pltpu.VMEM((CHUNK * WBLK,), jnp.int32),
            pltpu.VMEM((CHUNK * WBLK,), jnp.int32),
            pltpu.VMEM((CHUNK * WBLK, 128), jnp.float32),
            pltpu.VMEM((CHUNK * WBLK, 128), jnp.float32),
            pltpu.VMEM((2 * ABATCH,), jnp.int32),
            pltpu.VMEM((16,), jnp.float32),
            pltpu.VMEM((WBLK * 16, 128), jnp.float32),
            pltpu.SemaphoreType.DMA,
            pltpu.SemaphoreType.DMA,
            pltpu.SemaphoreType.DMA,
            pltpu.SemaphoreType.DMA,
        ],
        compiler_params=pltpu.CompilerParams(
            needs_layout_passes=False, use_tc_tiling_on_sc=False
        ),
    )
    o_word, o_pos, o_polar, o_position, o_aspect = k(
        ctx, posi, poli, psni, aspi, bnd,
        wt_phys, pt_phys, plt_phys, pst_phys,
    )
    word = o_word.reshape(B * L, WBLK * 128)[:, :DW].reshape(B, L, DW)
    pos = o_pos[:, :DP].reshape(B, L, DP)
    polar = o_polar[:, :DP].reshape(B, L, DP)
    position = o_position[:, :DP].reshape(B, L, DP)
    aspect_pool = o_aspect.reshape(B, WBLK * 128)[:, :DW].reshape(B, 1, DW)
    return (aspect_pool, word, pos, polar, position)


# speedup vs baseline: 1.7516x; 1.0143x over previous
import jax
import jax.numpy as jnp
from jax import lax
from jax.experimental import pallas as pl
from jax.experimental.pallas import tpu as pltpu
from jax.experimental.pallas import tpu_sc as plsc

B = 1024
L = 200
A = 8
DW = 300
DP = 50

NW = 32
RPW = (B * L) // NW
CHUNK = 128
NCH = RPW // CHUNK
SCHUNK = 320
NCHS = RPW // SCHUNK
ABATCH = B // NW
WBLK = 3

NPOS = 50
NPOLAR = 4
NPOSITION = 201
POS_REP = 256
POLAR_REP = 2048
POSITION_REP = 128


def _sc_body(ctx_i, pos_i, polar_i, position_i, asp_i, bnd_i,
             wt_phys, pt_phys, plt_phys, pst_phys,
             o_word, o_pos, o_polar, o_position, o_aspect,
             idx_v, idx3A, idx3B, bufA, bufB, bnd_v, len_v, apool_v,
             gsemA, gsemB, wsemA, wsemB):
    wid = lax.axis_index("s") * 2 + lax.axis_index("c")
    base = wid * RPW
    lanes = lax.iota(jnp.int32, 16)

    def build(idx3, c):
        off = c * CHUNK
        for t in range((CHUNK * WBLK) // 16):
            e = t * 16 + lanes
            k = e // WBLK
            cc = e - k * WBLK
            src = plsc.load_gather(idx_v, [off + k])
            idx3[pl.ds(t * 16, 16)] = src * WBLK + cc

    def wg(idx3, buf, sem):
        pltpu.async_copy(wt_phys.at[idx3], buf, sem)

    def wg_wait(idx3, buf, sem):
        pltpu.make_async_copy(wt_phys.at[idx3], buf, sem).wait()

    def ww(buf, c, sem):
        pltpu.async_copy(
            buf, o_word.at[pl.ds(WBLK * (base + c * CHUNK), WBLK * CHUNK)],
            sem)

    def ww_wait(buf, c, sem):
        pltpu.make_async_copy(
            buf, o_word.at[pl.ds(WBLK * (base + c * CHUNK), WBLK * CHUNK)],
            sem).wait()

    pltpu.sync_copy(ctx_i.at[pl.ds(base, RPW)], idx_v)
    build(idx3A, 0)
    wg(idx3A, bufA, gsemA)
    build(idx3B, 1)
    wg(idx3B, bufB, gsemB)
    wg_wait(idx3A, bufA, gsemA)
    ww(bufA, 0, wsemA)

    def wbody(i, carry):
        c = 2 * i
        build(idx3A, c)
        ww_wait(bufA, c - 2, wsemA)
        wg(idx3A, bufA, gsemA)
        wg_wait(idx3B, bufB, gsemB)
        ww(bufB, c - 1, wsemB)
        build(idx3B, c + 1)
        ww_wait(bufB, c - 1, wsemB)
        wg(idx3B, bufB, gsemB)
        wg_wait(idx3A, bufA, gsemA)
        ww(bufA, c, wsemA)
        return carry

    lax.fori_loop(1, NCH // 2, wbody, 0)
    wg_wait(idx3B, bufB, gsemB)
    ww(bufB, NCH - 1, wsemB)
    ww_wait(bufA, NCH - 2, wsemA)
    ww_wait(bufB, NCH - 1, wsemB)

    def small_phase(idx_hbm, table, out, n_rows, rep_mask, sidxA, sidxB):
        pltpu.sync_copy(idx_hbm.at[pl.ds(base, RPW)], idx_v)
        sbufA = bufA.at[pl.ds(0, SCHUNK)]
        sbufB = bufB.at[pl.ds(0, SCHUNK)]

        def sbuild(sidx, c):
            wskew = wid * ((rep_mask + 1) // NW)
            for t in range(SCHUNK // 16):
                iv = idx_v[pl.ds(c * SCHUNK + t * 16, 16)]
                spread = (t * 16 + lanes + wskew) & rep_mask
                sidx[pl.ds(t * 16, 16)] = iv + n_rows * spread

        def sg(sidx, slot, sem):
            pltpu.async_copy(table.at[sidx.at[pl.ds(0, SCHUNK)]], slot, sem)

        def sg_wait(sidx, slot, sem):
            pltpu.make_async_copy(
                table.at[sidx.at[pl.ds(0, SCHUNK)]], slot, sem).wait()

        def sw(slot, c, sem):
            pltpu.async_copy(slot, out.at[pl.ds(base + c * SCHUNK, SCHUNK)],
                             sem)

        def sw_wait(slot, c, sem):
            pltpu.make_async_copy(
                slot, out.at[pl.ds(base + c * SCHUNK, SCHUNK)], sem).wait()

        sbuild(sidxA, 0)
        sg(sidxA, sbufA, gsemA)
        sbuild(sidxB, 1)
        sg(sidxB, sbufB, gsemB)
        sg_wait(sidxA, sbufA, gsemA)
        sw(sbufA, 0, wsemA)

        def body(i, carry):
            c = 2 * i
            sbuild(sidxA, c)
            sw_wait(sbufA, c - 2, wsemA)
            sg(sidxA, sbufA, gsemA)
            sg_wait(sidxB, sbufB, gsemB)
            sw(sbufB, c - 1, wsemB)
            sbuild(sidxB, c + 1)
            sw_wait(sbufB, c - 1, wsemB)
            sg(sidxB, sbufB, gsemB)
            sg_wait(sidxA, sbufA, gsemA)
            sw(sbufA, c, wsemA)
            return carry

        lax.fori_loop(1, NCHS // 2, body, 0)
        sg_wait(sidxB, sbufB, gsemB)
        sw(sbufB, NCHS - 1, wsemB)
        sw_wait(sbufA, NCHS - 2, wsemA)
        sw_wait(sbufB, NCHS - 1, wsemB)

    small_phase(pos_i, pt_phys, o_pos, NPOS, POS_REP - 1, idx3A, idx3B)
    small_phase(polar_i, plt_phys, o_polar, NPOLAR, POLAR_REP - 1,
                idx3A, idx3B)
    small_phase(position_i, pst_phys, o_position, NPOSITION,
                POSITION_REP - 1, idx3A, idx3B)

    abase = wid * ABATCH
    pltpu.sync_copy(bnd_i.at[pl.ds(abase * 2, 2 * ABATCH)], bnd_v)
    for sub in range(2):
        bsub = abase + sub * 16
        pltpu.sync_copy(asp_i.at[pl.ds(bsub * A, 16 * A)],
                        idx_v.at[pl.ds(0, 16 * A)])
        build(idx3A, 0)
        wg(idx3A, bufA, gsemA)
        wg_wait(idx3A, bufA, gsemA)
        bidx = (sub * 16 + lanes) * 2
        b0 = plsc.load_gather(bnd_v, [bidx])
        b1 = plsc.load_gather(bnd_v, [bidx + 1])
        len_v[...] = (b1 - b0 + 1).astype(jnp.float32)

        def achunk(k, carry):
            j = k * 16 + lanes
            bl = j // DW
            dd = j - bl * DW
            blk = dd // 128
            col = dd - blk * 128
            acc = plsc.load_gather(bufA, [(bl * A) * WBLK + blk, col])
            for a in range(1, A):
                acc = acc + plsc.load_gather(
                    bufA, [(bl * A + a) * WBLK + blk, col])
            lenv = plsc.load_gather(len_v, [bl])
            plsc.store_scatter(apool_v, [bl * WBLK + blk, col], acc / lenv)
            return carry

        lax.fori_loop(0, (16 * DW) // 16, achunk, 0)
        pltpu.sync_copy(apool_v, o_aspect.at[pl.ds(WBLK * bsub, WBLK * 16)])


def kernel(word_table, pos_table, polar_table, position_table,
           context_indices, pos_indices, polar_indices, text_indices,
           position_indices, aspect_indices, aspect_boundary, target, len_s):
    wt_phys = jnp.pad(word_table, ((0, 0), (0, WBLK * 128 - DW))) \
                 .reshape(word_table.shape[0] * WBLK, 128)
    pt_phys = jnp.tile(jnp.pad(pos_table, ((0, 0), (0, 128 - DP))),
                       (POS_REP, 1))
    plt_phys = jnp.tile(jnp.pad(polar_table, ((0, 0), (0, 128 - DP))),
                        (POLAR_REP, 1))
    pst_phys = jnp.tile(jnp.pad(position_table, ((0, 0), (0, 128 - DP))),
                        (POSITION_REP, 1))

    ctx = context_indices.reshape(-1)
    posi = pos_indices.reshape(-1)
    poli = polar_indices.reshape(-1)
    psni = position_indices.reshape(-1)
    aspi = aspect_indices.reshape(-1)
    bnd = aspect_boundary.reshape(-1)

    k = pl.kernel(
        _sc_body,
        out_type=(
            jax.ShapeDtypeStruct((B * L * WBLK, 128), jnp.float32),
            jax.ShapeDtypeStruct((B * L, 128), jnp.float32),
            jax.ShapeDtypeStruct((B * L, 128), jnp.float32),
            jax.ShapeDtypeStruct((B * L, 128), jnp.float32),
            jax.ShapeDtypeStruct((B * WBLK, 128), jnp.float32),
        ),
        mesh=plsc.VectorSubcoreMesh(core_axis_name="c", subcore_axis_name="s"),
        scratch_types=[
            pltpu.VMEM((RPW,), jnp.int32),
            pltpu.VMEM((CHUNK * WBLK,), jnp.int32),
            pltpu.VMEM((CHUNK * WBLK,), jnp.int32),
            pltpu.VMEM((CHUNK * WBLK, 128), jnp.float32),
            pltpu.VMEM((CHUNK * WBLK, 128), jnp.float32),
            pltpu.VMEM((2 * ABATCH,), jnp.int32),
            pltpu.VMEM((16,), jnp.float32),
            pltpu.VMEM((WBLK * 16, 128), jnp.float32),
            pltpu.SemaphoreType.DMA,
            pltpu.SemaphoreType.DMA,
            pltpu.SemaphoreType.DMA,
            pltpu.SemaphoreType.DMA,
        ],
        compiler_params=pltpu.CompilerParams(
            needs_layout_passes=False, use_tc_tiling_on_sc=False
        ),
    )
    o_word, o_pos, o_polar, o_position, o_aspect = k(
        ctx, posi, poli, psni, aspi, bnd,
        wt_phys, pt_phys, plt_phys, pst_phys,
    )
    word = o_word.reshape(B * L, WBLK * 128)[:, :DW].reshape(B, L, DW)
    pos = o_pos[:, :DP].reshape(B, L, DP)
    polar = o_polar[:, :DP].reshape(B, L, DP)
    position = o_position[:, :DP].reshape(B, L, DP)
    aspect_pool = o_aspect.reshape(B, WBLK * 128)[:, :DW].reshape(B, 1, DW)
    return (aspect_pool, word, pos, polar, position)


# speedup vs baseline: 1.7952x; 1.0249x over previous
import jax
import jax.numpy as jnp
from jax import lax
from jax.experimental import pallas as pl
from jax.experimental.pallas import tpu as pltpu
from jax.experimental.pallas import tpu_sc as plsc

B = 1024
L = 200
A = 8
DW = 300
DP = 50

NW = 32
RPW = (B * L) // NW
CHUNK = 128
NCH = RPW // CHUNK
SCHUNK = 320
NCHS = RPW // SCHUNK
ABATCH = B // NW
WBLK = 3

NPOS = 50
NPOLAR = 4
NPOSITION = 201
POS_REP = 256
POLAR_REP = 2048
POSITION_REP = 128


def _sc_word_body(ctx_i, asp_i, bnd_i, wt_phys,
                  o_word, o_aspect,
                  idx_v, idx3A, idx3B, bufA, bufB, bnd_v, len_v, apool_v,
                  gsemA, gsemB, wsemA, wsemB):
    wid = lax.axis_index("s") * 2 + lax.axis_index("c")
    base = wid * RPW
    lanes = lax.iota(jnp.int32, 16)

    def build(idx3, c):
        off = c * CHUNK
        for t in range((CHUNK * WBLK) // 16):
            e = t * 16 + lanes
            k = e // WBLK
            cc = e - k * WBLK
            src = plsc.load_gather(idx_v, [off + k])
            idx3[pl.ds(t * 16, 16)] = src * WBLK + cc

    def wg(idx3, buf, sem):
        pltpu.async_copy(wt_phys.at[idx3], buf, sem)

    def wg_wait(idx3, buf, sem):
        pltpu.make_async_copy(wt_phys.at[idx3], buf, sem).wait()

    def ww(buf, c, sem):
        pltpu.async_copy(
            buf, o_word.at[pl.ds(WBLK * (base + c * CHUNK), WBLK * CHUNK)],
            sem)

    def ww_wait(buf, c, sem):
        pltpu.make_async_copy(
            buf, o_word.at[pl.ds(WBLK * (base + c * CHUNK), WBLK * CHUNK)],
            sem).wait()

    pltpu.sync_copy(ctx_i.at[pl.ds(base, RPW)], idx_v)
    build(idx3A, 0)
    wg(idx3A, bufA, gsemA)
    build(idx3B, 1)
    wg(idx3B, bufB, gsemB)
    wg_wait(idx3A, bufA, gsemA)
    ww(bufA, 0, wsemA)

    def wbody(i, carry):
        c = 2 * i
        build(idx3A, c)
        ww_wait(bufA, c - 2, wsemA)
        wg(idx3A, bufA, gsemA)
        wg_wait(idx3B, bufB, gsemB)
        ww(bufB, c - 1, wsemB)
        build(idx3B, c + 1)
        ww_wait(bufB, c - 1, wsemB)
        wg(idx3B, bufB, gsemB)
        wg_wait(idx3A, bufA, gsemA)
        ww(bufA, c, wsemA)
        return carry

    lax.fori_loop(1, NCH // 2, wbody, 0)
    wg_wait(idx3B, bufB, gsemB)
    ww(bufB, NCH - 1, wsemB)
    ww_wait(bufA, NCH - 2, wsemA)
    ww_wait(bufB, NCH - 1, wsemB)

    _aspect_tail(asp_i, bnd_i, wt_phys, o_aspect,
                 idx_v, idx3A, bufA, bnd_v, len_v, apool_v,
                 gsemA, wid, lanes)


def _sc_small_body(pos_i, polar_i, position_i,
                   pt_phys, plt_phys, pst_phys,
                   o_pos, o_polar, o_position,
                   idx_v, sidxA2, sidxB2, bufA, bufB,
                   gsemA, gsemB, wsemA, wsemB):
    wid = lax.axis_index("s") * 2 + lax.axis_index("c")
    base = wid * RPW
    lanes = lax.iota(jnp.int32, 16)

    def small_phase(idx_hbm, table, out, n_rows, rep_mask, sidxA, sidxB):
        pltpu.sync_copy(idx_hbm.at[pl.ds(base, RPW)], idx_v)
        sbufA = bufA.at[pl.ds(0, SCHUNK)]
        sbufB = bufB.at[pl.ds(0, SCHUNK)]

        def sbuild(sidx, c):
            wskew = wid * ((rep_mask + 1) // NW)
            for t in range(SCHUNK // 16):
                iv = idx_v[pl.ds(c * SCHUNK + t * 16, 16)]
                spread = (t * 16 + lanes + wskew) & rep_mask
                sidx[pl.ds(t * 16, 16)] = iv + n_rows * spread

        def sg(sidx, slot, sem):
            pltpu.async_copy(table.at[sidx.at[pl.ds(0, SCHUNK)]], slot, sem)

        def sg_wait(sidx, slot, sem):
            pltpu.make_async_copy(
                table.at[sidx.at[pl.ds(0, SCHUNK)]], slot, sem).wait()

        def sw(slot, c, sem):
            pltpu.async_copy(slot, out.at[pl.ds(base + c * SCHUNK, SCHUNK)],
                             sem)

        def sw_wait(slot, c, sem):
            pltpu.make_async_copy(
                slot, out.at[pl.ds(base + c * SCHUNK, SCHUNK)], sem).wait()

        sbuild(sidxA, 0)
        sg(sidxA, sbufA, gsemA)
        sbuild(sidxB, 1)
        sg(sidxB, sbufB, gsemB)
        sg_wait(sidxA, sbufA, gsemA)
        sw(sbufA, 0, wsemA)

        def body(i, carry):
            c = 2 * i
            sbuild(sidxA, c)
            sw_wait(sbufA, c - 2, wsemA)
            sg(sidxA, sbufA, gsemA)
            sg_wait(sidxB, sbufB, gsemB)
            sw(sbufB, c - 1, wsemB)
            sbuild(sidxB, c + 1)
            sw_wait(sbufB, c - 1, wsemB)
            sg(sidxB, sbufB, gsemB)
            sg_wait(sidxA, sbufA, gsemA)
            sw(sbufA, c, wsemA)
            return carry

        lax.fori_loop(1, NCHS // 2, body, 0)
        sg_wait(sidxB, sbufB, gsemB)
        sw(sbufB, NCHS - 1, wsemB)
        sw_wait(sbufA, NCHS - 2, wsemA)
        sw_wait(sbufB, NCHS - 1, wsemB)

    small_phase(pos_i, pt_phys, o_pos, NPOS, POS_REP - 1, sidxA2, sidxB2)
    small_phase(polar_i, plt_phys, o_polar, NPOLAR, POLAR_REP - 1,
                sidxA2, sidxB2)
    small_phase(position_i, pst_phys, o_position, NPOSITION,
                POSITION_REP - 1, sidxA2, sidxB2)


def _aspect_tail(asp_i, bnd_i, wt_phys, o_aspect,
                 idx_v, idx3A, bufA, bnd_v, len_v, apool_v, gsemA,
                 wid, lanes):
    def build(idx3, c):
        off = c * CHUNK
        for t in range((CHUNK * WBLK) // 16):
            e = t * 16 + lanes
            k = e // WBLK
            cc = e - k * WBLK
            src = plsc.load_gather(idx_v, [off + k])
            idx3[pl.ds(t * 16, 16)] = src * WBLK + cc

    def wg(idx3, buf, sem):
        pltpu.async_copy(wt_phys.at[idx3], buf, sem)

    def wg_wait(idx3, buf, sem):
        pltpu.make_async_copy(wt_phys.at[idx3], buf, sem).wait()

    abase = wid * ABATCH
    pltpu.sync_copy(bnd_i.at[pl.ds(abase * 2, 2 * ABATCH)], bnd_v)
    for sub in range(2):
        bsub = abase + sub * 16
        pltpu.sync_copy(asp_i.at[pl.ds(bsub * A, 16 * A)],
                        idx_v.at[pl.ds(0, 16 * A)])
        build(idx3A, 0)
        wg(idx3A, bufA, gsemA)
        wg_wait(idx3A, bufA, gsemA)
        bidx = (sub * 16 + lanes) * 2
        b0 = plsc.load_gather(bnd_v, [bidx])
        b1 = plsc.load_gather(bnd_v, [bidx + 1])
        len_v[...] = (b1 - b0 + 1).astype(jnp.float32)

        def achunk(k, carry):
            j = k * 16 + lanes
            bl = j // DW
            dd = j - bl * DW
            blk = dd // 128
            col = dd - blk * 128
            acc = plsc.load_gather(bufA, [(bl * A) * WBLK + blk, col])
            for a in range(1, A):
                acc = acc + plsc.load_gather(
                    bufA, [(bl * A + a) * WBLK + blk, col])
            lenv = plsc.load_gather(len_v, [bl])
            plsc.store_scatter(apool_v, [bl * WBLK + blk, col], acc / lenv)
            return carry

        lax.fori_loop(0, (16 * DW) // 16, achunk, 0)
        pltpu.sync_copy(apool_v, o_aspect.at[pl.ds(WBLK * bsub, WBLK * 16)])


def kernel(word_table, pos_table, polar_table, position_table,
           context_indices, pos_indices, polar_indices, text_indices,
           position_indices, aspect_indices, aspect_boundary, target, len_s):
    wt_phys = jnp.pad(word_table, ((0, 0), (0, WBLK * 128 - DW))) \
                 .reshape(word_table.shape[0] * WBLK, 128)
    pt_phys = jnp.tile(jnp.pad(pos_table, ((0, 0), (0, 128 - DP))),
                       (POS_REP, 1))
    plt_phys = jnp.tile(jnp.pad(polar_table, ((0, 0), (0, 128 - DP))),
                        (POLAR_REP, 1))
    pst_phys = jnp.tile(jnp.pad(position_table, ((0, 0), (0, 128 - DP))),
                        (POSITION_REP, 1))

    ctx = context_indices.reshape(-1)
    posi = pos_indices.reshape(-1)
    poli = polar_indices.reshape(-1)
    psni = position_indices.reshape(-1)
    aspi = aspect_indices.reshape(-1)
    bnd = aspect_boundary.reshape(-1)

    mesh = plsc.VectorSubcoreMesh(core_axis_name="c", subcore_axis_name="s")
    cp = pltpu.CompilerParams(
        needs_layout_passes=False, use_tc_tiling_on_sc=False
    )
    k_small = pl.kernel(
        _sc_small_body,
        out_type=(
            jax.ShapeDtypeStruct((B * L, 128), jnp.float32),
            jax.ShapeDtypeStruct((B * L, 128), jnp.float32),
            jax.ShapeDtypeStruct((B * L, 128), jnp.float32),
        ),
        mesh=mesh,
        scratch_types=[
            pltpu.VMEM((RPW,), jnp.int32),
            pltpu.VMEM((SCHUNK,), jnp.int32),
            pltpu.VMEM((SCHUNK,), jnp.int32),
            pltpu.VMEM((SCHUNK, 128), jnp.float32),
            pltpu.VMEM((SCHUNK, 128), jnp.float32),
            pltpu.SemaphoreType.DMA,
            pltpu.SemaphoreType.DMA,
            pltpu.SemaphoreType.DMA,
            pltpu.SemaphoreType.DMA,
        ],
        compiler_params=cp,
    )
    k_word = pl.kernel(
        _sc_word_body,
        out_type=(
            jax.ShapeDtypeStruct((B * L * WBLK, 128), jnp.float32),
            jax.ShapeDtypeStruct((B * WBLK, 128), jnp.float32),
        ),
        mesh=mesh,
        scratch_types=[
            pltpu.VMEM((RPW,), jnp.int32),
            pltpu.VMEM((CHUNK * WBLK,), jnp.int32),
            pltpu.VMEM((CHUNK * WBLK,), jnp.int32),
            pltpu.VMEM((CHUNK * WBLK, 128), jnp.float32),
            pltpu.VMEM((CHUNK * WBLK, 128), jnp.float32),
            pltpu.VMEM((2 * ABATCH,), jnp.int32),
            pltpu.VMEM((16,), jnp.float32),
            pltpu.VMEM((WBLK * 16, 128), jnp.float32),
            pltpu.SemaphoreType.DMA,
            pltpu.SemaphoreType.DMA,
            pltpu.SemaphoreType.DMA,
            pltpu.SemaphoreType.DMA,
        ],
        compiler_params=cp,
    )
    o_pos, o_polar, o_position = k_small(
        posi, poli, psni, pt_phys, plt_phys, pst_phys)
    o_word, o_aspect = k_word(ctx, aspi, bnd, wt_phys)
    word = o_word.reshape(B * L, WBLK * 128)[:, :DW].reshape(B, L, DW)
    pos = o_pos[:, :DP].reshape(B, L, DP)
    polar = o_polar[:, :DP].reshape(B, L, DP)
    position = o_position[:, :DP].reshape(B, L, DP)
    aspect_pool = o_aspect.reshape(B, WBLK * 128)[:, :DW].reshape(B, 1, DW)
    return (aspect_pool, word, pos, polar, position)
